# baseline (speedup 1.0000x reference)
import jax
import jax.numpy as jnp
from jax import lax
from jax.experimental import pallas as pl
from jax.experimental.pallas import tpu as pltpu
from jax.experimental.pallas import tpu_sc as plsc

_B, _H, _W, _C = 4, 32, 32, 768
_NROIS = 32
_NBINS = 21
_NC, _NS = 2, 16
_NW = _NC * _NS
_CHUNKS = _NW // _B
_CPW = _C // _CHUNKS
_G = _CPW // 16


def _rhe(v):
    f = v.astype(jnp.int32)
    d = v - f.astype(jnp.float32)
    half = jnp.where(d == 0.5, f & 1, 0)
    return f + jnp.where(d > 0.5, 1, half)


def _vextract(vec, lane):
    idx = lax.iota(jnp.int32, 16)
    return jnp.max(jnp.where(idx == lane, vec, jnp.float32(-jnp.inf)))


def _body(img_hbm, rois_hbm, out_hbm, img_v, rois_v, res_v):
    wid = lax.axis_index("s") * _NC + lax.axis_index("c")
    b = wid // _CHUNKS
    c0 = (wid % _CHUNKS) * _CPW
    pltpu.sync_copy(img_hbm.at[b, :, :, pl.ds(c0, _CPW)], img_v)
    pltpu.sync_copy(rois_hbm.at[b], rois_v)

    neg = jnp.full((16,), -jnp.inf, jnp.float32)

    def roi_body(r, carry):
        lanes = lax.iota(jnp.int32, 16)
        v = plsc.load_gather(rois_v, [jnp.full((16,), r, jnp.int32), lanes & 3])
        x = _vextract(v, 0)
        y = _vextract(v, 1)
        w = _vextract(v, 2)
        h = _vextract(v, 3)
        ex = [_rhe(x + jnp.float32(j * 0.25) * h) for j in range(5)]
        ey = [_rhe(y + jnp.float32(j * 0.25) * w) for j in range(5)]

        for ix in range(4):
            xs = ex[ix]
            wx = ex[ix + 1] - xs
            for WX in (2, 3, 4, 5):
                @pl.when(wx == WX)
                def _bins(_ix=ix, _xs=xs, _WX=WX):
                    for jy in range(4):
                        def ystep(yy, accs):
                            out = accs
                            for dx in range(_WX):
                                out = tuple(
                                    jnp.maximum(
                                        out[g],
                                        img_v[yy, _xs + dx, pl.ds(16 * g, 16)])
                                    for g in range(_G)
                                )
                            return out
                        accs = lax.fori_loop(ey[jy], ey[jy + 1], ystep,
                                             (neg,) * _G)
                        for g in range(_G):
                            res_v[5 + _ix * 4 + jy, pl.ds(16 * g, 16)] = accs[g]

        for i2 in range(2):
            for j2 in range(2):
                kc = 1 + i2 * 2 + j2
                for g in range(_G):
                    s = pl.ds(16 * g, 16)
                    m0 = jnp.maximum(res_v[5 + (2 * i2) * 4 + 2 * j2, s],
                                     res_v[5 + (2 * i2) * 4 + 2 * j2 + 1, s])
                    m1 = jnp.maximum(res_v[5 + (2 * i2 + 1) * 4 + 2 * j2, s],
                                     res_v[5 + (2 * i2 + 1) * 4 + 2 * j2 + 1, s])
                    res_v[kc, s] = jnp.maximum(m0, m1)
        for g in range(_G):
            s = pl.ds(16 * g, 16)
            res_v[0, s] = jnp.maximum(
                jnp.maximum(res_v[1, s], res_v[2, s]),
                jnp.maximum(res_v[3, s], res_v[4, s]))

        pltpu.sync_copy(res_v, out_hbm.at[b, r, :, pl.ds(c0, _CPW)])
        return carry

    lax.fori_loop(0, _NROIS, roi_body, 0)


@jax.jit
def kernel(img, rois):
    fn = pl.kernel(
        _body,
        out_type=jax.ShapeDtypeStruct((_B, _NROIS, _NBINS, _C), jnp.float32),
        mesh=plsc.VectorSubcoreMesh(core_axis_name="c", subcore_axis_name="s",
                                    num_cores=_NC, num_subcores=_NS),
        compiler_params=pltpu.CompilerParams(use_tc_tiling_on_sc=False,
                                             needs_layout_passes=False),
        scratch_types=[
            pltpu.VMEM((_H, _W, _CPW), jnp.float32),
            pltpu.VMEM((_NROIS, 4), jnp.float32),
            pltpu.VMEM((_NBINS, _CPW), jnp.float32),
        ],
    )
    return fn(img, rois).reshape(_B, _NROIS, _NBINS * _C)


# speedup vs baseline: 1.5911x; 1.5911x over previous
import jax
import jax.numpy as jnp
from jax import lax
from jax.experimental import pallas as pl
from jax.experimental.pallas import tpu as pltpu
from jax.experimental.pallas import tpu_sc as plsc

_B, _H, _W, _C = 4, 32, 32, 768
_NROIS = 32
_NBINS = 21
_NC, _NS = 2, 16
_NW = _NC * _NS
_CHUNKS = _NW // _B
_CPW = _C // _CHUNKS
_G = _CPW // 16


def _rhe(v):
    f = v.astype(jnp.int32)
    d = v - f.astype(jnp.float32)
    half = jnp.where(d == 0.5, f & 1, 0)
    return f + jnp.where(d > 0.5, 1, half)


def _vextract(vec, lane):
    idx = lax.iota(jnp.int32, 16)
    return jnp.max(jnp.where(idx == lane, vec, jnp.float32(-jnp.inf)))


def _body(img_hbm, rois_hbm, out_hbm, img_v, rois_v, res_v):
    wid = lax.axis_index("s") * _NC + lax.axis_index("c")
    b = wid // _CHUNKS
    c0 = (wid % _CHUNKS) * _CPW
    for j in range(3):
        cj = c0 + 32 * j
        pltpu.sync_copy(
            img_hbm.at[b, :, :, cj // 128, :, pl.ds(cj % 128, 32)],
            img_v.at[:, :, :, pl.ds(32 * j, 32)])
    pltpu.sync_copy(rois_hbm.at[b], rois_v)

    neg = jnp.full((16,), -jnp.inf, jnp.float32)

    def roi_body(r, carry):
        lanes = lax.iota(jnp.int32, 16)
        v = plsc.load_gather(rois_v, [jnp.full((16,), r, jnp.int32), lanes & 3])
        x = _vextract(v, 0)
        y = _vextract(v, 1)
        w = _vextract(v, 2)
        h = _vextract(v, 3)
        ex = [_rhe(x + jnp.float32(j * 0.25) * h) for j in range(5)]
        ey = [_rhe(y + jnp.float32(j * 0.25) * w) for j in range(5)]

        for ix in range(4):
            for jy in range(4):
                def ystep(yy, accs, _ix=ix):
                    def xstep(xx, a):
                        return tuple(
                            jnp.maximum(
                                a[g],
                                img_v[yy, xx >> 3, xx & 7, pl.ds(16 * g, 16)])
                            for g in range(_G)
                        )
                    return lax.fori_loop(ex[_ix], ex[_ix + 1], xstep, accs)
                accs = lax.fori_loop(ey[jy], ey[jy + 1], ystep, (neg,) * _G)
                for g in range(_G):
                    res_v[5 + ix * 4 + jy, pl.ds(16 * g, 16)] = accs[g]

        for i2 in range(2):
            for j2 in range(2):
                kc = 1 + i2 * 2 + j2
                for g in range(_G):
                    s = pl.ds(16 * g, 16)
                    m0 = jnp.maximum(res_v[5 + (2 * i2) * 4 + 2 * j2, s],
                                     res_v[5 + (2 * i2) * 4 + 2 * j2 + 1, s])
                    m1 = jnp.maximum(res_v[5 + (2 * i2 + 1) * 4 + 2 * j2, s],
                                     res_v[5 + (2 * i2 + 1) * 4 + 2 * j2 + 1, s])
                    res_v[kc, s] = jnp.maximum(m0, m1)
        for g in range(_G):
            s = pl.ds(16 * g, 16)
            res_v[0, s] = jnp.maximum(
                jnp.maximum(res_v[1, s], res_v[2, s]),
                jnp.maximum(res_v[3, s], res_v[4, s]))

        pltpu.sync_copy(res_v, out_hbm.at[b, r, :, pl.ds(c0, _CPW)])
        return carry

    lax.fori_loop(0, _NROIS, roi_body, 0)


@jax.jit
def kernel(img, rois):
    fn = pl.kernel(
        _body,
        out_type=jax.ShapeDtypeStruct((_B, _NROIS, _NBINS, _C), jnp.float32),
        mesh=plsc.VectorSubcoreMesh(core_axis_name="c", subcore_axis_name="s",
                                    num_cores=_NC, num_subcores=_NS),
        compiler_params=pltpu.CompilerParams(use_tc_tiling_on_sc=False,
                                             needs_layout_passes=False),
        scratch_types=[
            pltpu.VMEM((_H, _W // 8, 8, _CPW), jnp.float32),
            pltpu.VMEM((_NROIS, 4), jnp.float32),
            pltpu.VMEM((_NBINS, _CPW), jnp.float32),
        ],
    )
    img6 = jnp.transpose(img.reshape(_B, _H, _W // 8, 8, _C // 128, 128),
                         (0, 1, 2, 4, 3, 5))
    return fn(img6, rois).reshape(_B, _NROIS, _NBINS * _C)


# speedup vs baseline: 1.5987x; 1.0048x over previous
import jax
import jax.numpy as jnp
from jax import lax
from jax.experimental import pallas as pl
from jax.experimental.pallas import tpu as pltpu
from jax.experimental.pallas import tpu_sc as plsc

_B, _H, _W, _C = 4, 32, 32, 768
_NROIS = 32
_NBINS = 21
_NC, _NS = 2, 16
_NW = _NC * _NS
_CHUNKS = _NW // _B
_CPW = _C // _CHUNKS
_G = _CPW // 16


def _rhe(v):
    f = v.astype(jnp.int32)
    d = v - f.astype(jnp.float32)
    half = jnp.where(d == 0.5, f & 1, 0)
    return f + jnp.where(d > 0.5, 1, half)


def _vextract(vec, lane):
    idx = lax.iota(jnp.int32, 16)
    return jnp.max(jnp.where(idx == lane, vec, jnp.float32(-jnp.inf)))


def _body(img_hbm, rois_hbm, out_hbm, img_v, rois_v, res_v):
    wid = lax.axis_index("s") * _NC + lax.axis_index("c")
    b = wid // _CHUNKS
    c0 = (wid % _CHUNKS) * _CPW
    ct0 = c0 // 128
    l0 = c0 % 128

    @pl.when(l0 == 0)
    def _c0():
        pltpu.sync_copy(img_hbm.at[b, :, :, ct0, :, pl.ds(0, 96)], img_v)

    @pl.when(l0 == 32)
    def _c32():
        pltpu.sync_copy(img_hbm.at[b, :, :, ct0, :, pl.ds(32, 96)], img_v)

    @pl.when(l0 == 64)
    def _c64():
        pltpu.sync_copy(img_hbm.at[b, :, :, ct0, :, pl.ds(64, 64)],
                        img_v.at[:, :, :, pl.ds(0, 64)])
        pltpu.sync_copy(img_hbm.at[b, :, :, ct0 + 1, :, pl.ds(0, 32)],
                        img_v.at[:, :, :, pl.ds(64, 32)])

    @pl.when(l0 == 96)
    def _c96():
        pltpu.sync_copy(img_hbm.at[b, :, :, ct0, :, pl.ds(96, 32)],
                        img_v.at[:, :, :, pl.ds(0, 32)])
        pltpu.sync_copy(img_hbm.at[b, :, :, ct0 + 1, :, pl.ds(0, 64)],
                        img_v.at[:, :, :, pl.ds(32, 64)])
    pltpu.sync_copy(rois_hbm.at[b], rois_v)

    neg = jnp.full((16,), -jnp.inf, jnp.float32)

    def roi_body(r, carry):
        lanes = lax.iota(jnp.int32, 16)
        v = plsc.load_gather(rois_v, [jnp.full((16,), r, jnp.int32), lanes & 3])
        x = _vextract(v, 0)
        y = _vextract(v, 1)
        w = _vextract(v, 2)
        h = _vextract(v, 3)
        ex = [_rhe(x + jnp.float32(j * 0.25) * h) for j in range(5)]
        ey = [_rhe(y + jnp.float32(j * 0.25) * w) for j in range(5)]

        for ix in range(4):
            for jy in range(4):
                def ystep(yy, accs, _ix=ix):
                    def xstep(xx, a):
                        return tuple(
                            jnp.maximum(
                                a[g],
                                img_v[yy, xx >> 3, xx & 7, pl.ds(16 * g, 16)])
                            for g in range(_G)
                        )
                    return lax.fori_loop(ex[_ix], ex[_ix + 1], xstep, accs)
                accs = lax.fori_loop(ey[jy], ey[jy + 1], ystep, (neg,) * _G)
                for g in range(_G):
                    res_v[5 + ix * 4 + jy, pl.ds(16 * g, 16)] = accs[g]

        for i2 in range(2):
            for j2 in range(2):
                kc = 1 + i2 * 2 + j2
                for g in range(_G):
                    s = pl.ds(16 * g, 16)
                    m0 = jnp.maximum(res_v[5 + (2 * i2) * 4 + 2 * j2, s],
                                     res_v[5 + (2 * i2) * 4 + 2 * j2 + 1, s])
                    m1 = jnp.maximum(res_v[5 + (2 * i2 + 1) * 4 + 2 * j2, s],
                                     res_v[5 + (2 * i2 + 1) * 4 + 2 * j2 + 1, s])
                    res_v[kc, s] = jnp.maximum(m0, m1)
        for g in range(_G):
            s = pl.ds(16 * g, 16)
            res_v[0, s] = jnp.maximum(
                jnp.maximum(res_v[1, s], res_v[2, s]),
                jnp.maximum(res_v[3, s], res_v[4, s]))

        pltpu.sync_copy(res_v, out_hbm.at[b, r, :, pl.ds(c0, _CPW)])
        return carry

    lax.fori_loop(0, _NROIS, roi_body, 0)


@jax.jit
def kernel(img, rois):
    fn = pl.kernel(
        _body,
        out_type=jax.ShapeDtypeStruct((_B, _NROIS, _NBINS, _C), jnp.float32),
        mesh=plsc.VectorSubcoreMesh(core_axis_name="c", subcore_axis_name="s",
                                    num_cores=_NC, num_subcores=_NS),
        compiler_params=pltpu.CompilerParams(use_tc_tiling_on_sc=False,
                                             needs_layout_passes=False),
        scratch_types=[
            pltpu.VMEM((_H, _W // 8, 8, _CPW), jnp.float32),
            pltpu.VMEM((_NROIS, 4), jnp.float32),
            pltpu.VMEM((_NBINS, _CPW), jnp.float32),
        ],
    )
    img6 = jnp.transpose(img.reshape(_B, _H, _W // 8, 8, _C // 128, 128),
                         (0, 1, 2, 4, 3, 5))
    return fn(img6, rois).reshape(_B, _NROIS, _NBINS * _C)


# speedup vs baseline: 1.6503x; 1.0323x over previous
import jax
import jax.numpy as jnp
from jax import lax
from jax.experimental import pallas as pl
from jax.experimental.pallas import tpu as pltpu
from jax.experimental.pallas import tpu_sc as plsc

_B, _H, _W, _C = 4, 32, 32, 768
_NROIS = 32
_NBINS = 21
_NC, _NS = 2, 16
_NW = _NC * _NS
_CHUNKS = _NW // _B
_CPW = _C // _CHUNKS
_G = _CPW // 16


def _rhe(v):
    f = v.astype(jnp.int32)
    d = v - f.astype(jnp.float32)
    half = jnp.where(d == 0.5, f & 1, 0)
    return f + jnp.where(d > 0.5, 1, half)


def _vextract(vec, lane):
    idx = lax.iota(jnp.int32, 16)
    return jnp.max(jnp.where(idx == lane, vec, jnp.float32(-jnp.inf)))


def _body(img_hbm, rois_hbm, out_hbm, img_v, rois_v, res_v):
    wid = lax.axis_index("s") * _NC + lax.axis_index("c")
    b = wid // _CHUNKS
    c0 = (wid % _CHUNKS) * _CPW
    ct0 = c0 // 128
    l0 = c0 % 128

    @pl.when(l0 == 0)
    def _c0():
        for xt in range(4):
            pltpu.sync_copy(img_hbm.at[b, :, xt, ct0, :, pl.ds(0, 96)],
                            img_v.at[:, pl.ds(8 * xt, 8), :])

    @pl.when(l0 == 32)
    def _c32():
        for xt in range(4):
            pltpu.sync_copy(img_hbm.at[b, :, xt, ct0, :, pl.ds(32, 96)],
                            img_v.at[:, pl.ds(8 * xt, 8), :])

    @pl.when(l0 == 64)
    def _c64():
        for xt in range(4):
            pltpu.sync_copy(img_hbm.at[b, :, xt, ct0, :, pl.ds(64, 64)],
                            img_v.at[:, pl.ds(8 * xt, 8), pl.ds(0, 64)])
            pltpu.sync_copy(img_hbm.at[b, :, xt, ct0 + 1, :, pl.ds(0, 32)],
                            img_v.at[:, pl.ds(8 * xt, 8), pl.ds(64, 32)])

    @pl.when(l0 == 96)
    def _c96():
        for xt in range(4):
            pltpu.sync_copy(img_hbm.at[b, :, xt, ct0, :, pl.ds(96, 32)],
                            img_v.at[:, pl.ds(8 * xt, 8), pl.ds(0, 32)])
            pltpu.sync_copy(img_hbm.at[b, :, xt, ct0 + 1, :, pl.ds(0, 64)],
                            img_v.at[:, pl.ds(8 * xt, 8), pl.ds(32, 64)])
    pltpu.sync_copy(rois_hbm.at[b], rois_v)

    neg = jnp.full((16,), -jnp.inf, jnp.float32)

    def roi_body(r, carry):
        lanes = lax.iota(jnp.int32, 16)
        v = plsc.load_gather(rois_v, [jnp.full((16,), r, jnp.int32), lanes & 3])
        x = _vextract(v, 0)
        y = _vextract(v, 1)
        w = _vextract(v, 2)
        h = _vextract(v, 3)
        ex = [_rhe(x + jnp.float32(j * 0.25) * h) for j in range(5)]
        ey = [_rhe(y + jnp.float32(j * 0.25) * w) for j in range(5)]

        for ix in range(4):
            for jy in range(4):
                def ystep(yy, accs, _ix=ix):
                    def xstep(xx, a):
                        return tuple(
                            jnp.maximum(
                                a[g],
                                img_v[yy, xx, pl.ds(16 * g, 16)])
                            for g in range(_G)
                        )
                    return lax.fori_loop(ex[_ix], ex[_ix + 1], xstep, accs)
                accs = lax.fori_loop(ey[jy], ey[jy + 1], ystep, (neg,) * _G)
                for g in range(_G):
                    res_v[5 + ix * 4 + jy, pl.ds(16 * g, 16)] = accs[g]

        for i2 in range(2):
            for j2 in range(2):
                kc = 1 + i2 * 2 + j2
                for g in range(_G):
                    s = pl.ds(16 * g, 16)
                    m0 = jnp.maximum(res_v[5 + (2 * i2) * 4 + 2 * j2, s],
                                     res_v[5 + (2 * i2) * 4 + 2 * j2 + 1, s])
                    m1 = jnp.maximum(res_v[5 + (2 * i2 + 1) * 4 + 2 * j2, s],
                                     res_v[5 + (2 * i2 + 1) * 4 + 2 * j2 + 1, s])
                    res_v[kc, s] = jnp.maximum(m0, m1)
        for g in range(_G):
            s = pl.ds(16 * g, 16)
            res_v[0, s] = jnp.maximum(
                jnp.maximum(res_v[1, s], res_v[2, s]),
                jnp.maximum(res_v[3, s], res_v[4, s]))

        pltpu.sync_copy(res_v, out_hbm.at[b, r, :, pl.ds(c0, _CPW)])
        return carry

    lax.fori_loop(0, _NROIS, roi_body, 0)


@jax.jit
def kernel(img, rois):
    fn = pl.kernel(
        _body,
        out_type=jax.ShapeDtypeStruct((_B, _NROIS, _NBINS, _C), jnp.float32),
        mesh=plsc.VectorSubcoreMesh(core_axis_name="c", subcore_axis_name="s",
                                    num_cores=_NC, num_subcores=_NS),
        compiler_params=pltpu.CompilerParams(use_tc_tiling_on_sc=False,
                                             needs_layout_passes=False),
        scratch_types=[
            pltpu.VMEM((_H, _W, _CPW), jnp.float32),
            pltpu.VMEM((_NROIS, 4), jnp.float32),
            pltpu.VMEM((_NBINS, _CPW), jnp.float32),
        ],
    )
    img6 = jnp.transpose(img.reshape(_B, _H, _W // 8, 8, _C // 128, 128),
                         (0, 1, 2, 4, 3, 5))
    return fn(img6, rois).reshape(_B, _NROIS, _NBINS * _C)


# speedup vs baseline: 1.7213x; 1.0430x over previous
import jax
import jax.numpy as jnp
from jax import lax
from jax.experimental import pallas as pl
from jax.experimental.pallas import tpu as pltpu
from jax.experimental.pallas import tpu_sc as plsc

_B, _H, _W, _C = 4, 32, 32, 768
_NROIS = 32
_NBINS = 21
_NC, _NS = 2, 16
_NW = _NC * _NS
_CHUNKS = _NW // _B
_CPW = _C // _CHUNKS
_G = _CPW // 16


def _rhe(v):
    f = v.astype(jnp.int32)
    d = v - f.astype(jnp.float32)
    half = jnp.where(d == 0.5, f & 1, 0)
    return f + jnp.where(d > 0.5, 1, half)


def _vextract(vec, lane):
    idx = lax.iota(jnp.int32, 16)
    return jnp.max(jnp.where(idx == lane, vec, jnp.float32(-jnp.inf)))


def _body(img_hbm, rois_hbm, out_hbm, img_v, rois_v, res_v, dsem):
    wid = lax.axis_index("s") * _NC + lax.axis_index("c")
    b = wid // _CHUNKS
    c0 = (wid % _CHUNKS) * _CPW
    ct0 = c0 // 128
    l0 = c0 % 128

    @pl.when(l0 == 0)
    def _c0():
        for xt in range(4):
            pltpu.async_copy(img_hbm.at[b, :, xt, ct0, :, pl.ds(0, 96)],
                             img_v.at[:, pl.ds(8 * xt, 8), :], dsem)
        for xt in range(4):
            pltpu.make_async_copy(img_hbm.at[b, :, xt, ct0, :, pl.ds(0, 96)],
                                  img_v.at[:, pl.ds(8 * xt, 8), :], dsem).wait()

    @pl.when(l0 == 32)
    def _c32():
        for xt in range(4):
            pltpu.async_copy(img_hbm.at[b, :, xt, ct0, :, pl.ds(32, 96)],
                             img_v.at[:, pl.ds(8 * xt, 8), :], dsem)
        for xt in range(4):
            pltpu.make_async_copy(img_hbm.at[b, :, xt, ct0, :, pl.ds(32, 96)],
                                  img_v.at[:, pl.ds(8 * xt, 8), :], dsem).wait()

    @pl.when(l0 == 64)
    def _c64():
        for xt in range(4):
            pltpu.async_copy(img_hbm.at[b, :, xt, ct0, :, pl.ds(64, 64)],
                             img_v.at[:, pl.ds(8 * xt, 8), pl.ds(0, 64)], dsem)
            pltpu.async_copy(img_hbm.at[b, :, xt, ct0 + 1, :, pl.ds(0, 32)],
                             img_v.at[:, pl.ds(8 * xt, 8), pl.ds(64, 32)], dsem)
        for xt in range(4):
            pltpu.make_async_copy(img_hbm.at[b, :, xt, ct0, :, pl.ds(64, 64)],
                                  img_v.at[:, pl.ds(8 * xt, 8), pl.ds(0, 64)], dsem).wait()
            pltpu.make_async_copy(img_hbm.at[b, :, xt, ct0 + 1, :, pl.ds(0, 32)],
                                  img_v.at[:, pl.ds(8 * xt, 8), pl.ds(64, 32)], dsem).wait()

    @pl.when(l0 == 96)
    def _c96():
        for xt in range(4):
            pltpu.async_copy(img_hbm.at[b, :, xt, ct0, :, pl.ds(96, 32)],
                             img_v.at[:, pl.ds(8 * xt, 8), pl.ds(0, 32)], dsem)
            pltpu.async_copy(img_hbm.at[b, :, xt, ct0 + 1, :, pl.ds(0, 64)],
                             img_v.at[:, pl.ds(8 * xt, 8), pl.ds(32, 64)], dsem)
        for xt in range(4):
            pltpu.make_async_copy(img_hbm.at[b, :, xt, ct0, :, pl.ds(96, 32)],
                                  img_v.at[:, pl.ds(8 * xt, 8), pl.ds(0, 32)], dsem).wait()
            pltpu.make_async_copy(img_hbm.at[b, :, xt, ct0 + 1, :, pl.ds(0, 64)],
                                  img_v.at[:, pl.ds(8 * xt, 8), pl.ds(32, 64)], dsem).wait()
    pltpu.sync_copy(rois_hbm.at[b], rois_v)

    neg = jnp.full((16,), -jnp.inf, jnp.float32)

    def roi_body(r, carry):
        lanes = lax.iota(jnp.int32, 16)
        v = plsc.load_gather(rois_v, [jnp.full((16,), r, jnp.int32), lanes & 3])
        x = _vextract(v, 0)
        y = _vextract(v, 1)
        w = _vextract(v, 2)
        h = _vextract(v, 3)
        ex = [_rhe(x + jnp.float32(j * 0.25) * h) for j in range(5)]
        ey = [_rhe(y + jnp.float32(j * 0.25) * w) for j in range(5)]

        for ix in range(4):
            for jy in range(4):
                def ystep(yy, accs, _ix=ix):
                    def xstep(xx, a):
                        return tuple(
                            jnp.maximum(
                                a[g],
                                img_v[yy, xx, pl.ds(16 * g, 16)])
                            for g in range(_G)
                        )
                    return lax.fori_loop(ex[_ix], ex[_ix + 1], xstep, accs)
                accs = lax.fori_loop(ey[jy], ey[jy + 1], ystep, (neg,) * _G)
                for g in range(_G):
                    res_v[5 + ix * 4 + jy, pl.ds(16 * g, 16)] = accs[g]

        for i2 in range(2):
            for j2 in range(2):
                kc = 1 + i2 * 2 + j2
                for g in range(_G):
                    s = pl.ds(16 * g, 16)
                    m0 = jnp.maximum(res_v[5 + (2 * i2) * 4 + 2 * j2, s],
                                     res_v[5 + (2 * i2) * 4 + 2 * j2 + 1, s])
                    m1 = jnp.maximum(res_v[5 + (2 * i2 + 1) * 4 + 2 * j2, s],
                                     res_v[5 + (2 * i2 + 1) * 4 + 2 * j2 + 1, s])
                    res_v[kc, s] = jnp.maximum(m0, m1)
        for g in range(_G):
            s = pl.ds(16 * g, 16)
            res_v[0, s] = jnp.maximum(
                jnp.maximum(res_v[1, s], res_v[2, s]),
                jnp.maximum(res_v[3, s], res_v[4, s]))

        pltpu.sync_copy(res_v, out_hbm.at[b, r, :, pl.ds(c0, _CPW)])
        return carry

    lax.fori_loop(0, _NROIS, roi_body, 0)


@jax.jit
def kernel(img, rois):
    fn = pl.kernel(
        _body,
        out_type=jax.ShapeDtypeStruct((_B, _NROIS, _NBINS, _C), jnp.float32),
        mesh=plsc.VectorSubcoreMesh(core_axis_name="c", subcore_axis_name="s",
                                    num_cores=_NC, num_subcores=_NS),
        compiler_params=pltpu.CompilerParams(use_tc_tiling_on_sc=False,
                                             needs_layout_passes=False),
        scratch_types=[
            pltpu.VMEM((_H, _W, _CPW), jnp.float32),
            pltpu.VMEM((_NROIS, 4), jnp.float32),
            pltpu.VMEM((_NBINS, _CPW), jnp.float32),
            pltpu.SemaphoreType.DMA,
        ],
    )
    img6 = jnp.transpose(img.reshape(_B, _H, _W // 8, 8, _C // 128, 128),
                         (0, 1, 2, 4, 3, 5))
    return fn(img6, rois).reshape(_B, _NROIS, _NBINS * _C)
